# baseline (device time: 98166 ns/iter reference)
import jax
import jax.numpy as jnp
from jax import lax
from jax.experimental import pallas as pl
from jax.experimental.pallas import tpu as pltpu

N_DEV = 16
M = 512
NC = 512
H = NC // 2
K = 4
RS = M // K


def kernel(x):
    x2 = x.reshape(M, N_DEV * NC)

    def body(x_ref, out_ref, cw_ref, ccw_ref,
             cw_send, cw_recv, ccw_send, ccw_recv):
        my = lax.axis_index("i")

        def pos_of(l):
            q = l % 4
            z = l // 4
            return q * 4 + jnp.where(q % 2 == 0, z, 3 - z)

        def id_of(t):
            t = t % N_DEV
            q = t // 4
            zp = t % 4
            z = jnp.where(q % 2 == 0, zp, 3 - zp)
            return z * 4 + q

        p = pos_of(my)
        left = id_of(p - 1)
        right = id_of(p + 1)

        barrier_sem = pltpu.get_barrier_semaphore()
        for nbr in (left, right):
            pl.semaphore_signal(
                barrier_sem, inc=1,
                device_id=(nbr,), device_id_type=pl.DeviceIdType.MESH,
            )
        pl.semaphore_wait(barrier_sem, 2)

        def make(d, s, r):
            ref, ssem, rsem, tgt = (
                (cw_ref, cw_send, cw_recv, right) if d == 0
                else (ccw_ref, ccw_send, ccw_recv, left)
            )
            rows = pl.ds(r * RS, RS)
            return pltpu.make_async_remote_copy(
                src_ref=ref.at[s, rows, :],
                dst_ref=ref.at[s + 1, rows, :],
                send_sem=ssem.at[s, r],
                recv_sem=rsem.at[s, r],
                device_id=(tgt,),
                device_id_type=pl.DeviceIdType.MESH,
            )

        descs = {
            (d, s, r): make(d, s, r)
            for d in (0, 1) for s in range(N_DEV - 1) for r in range(K)
        }

        cw_ref[0, :, :] = x_ref[:, pl.ds(id_of(p - 1) * NC, H)]
        ccw_ref[0, :, :] = x_ref[:, pl.ds(id_of(p + 1) * NC + H, H)]
        for r in range(K):
            descs[(0, 0, r)].start()
            descs[(1, 0, r)].start()

        for s in range(N_DEV - 1):
            c_cw = id_of(p - 2 - s)
            c_ccw = id_of(p + 2 + s)
            for r in range(K):
                rows = pl.ds(r * RS, RS)
                for d, ref, c, off in (
                    (0, cw_ref, c_cw, 0),
                    (1, ccw_ref, c_ccw, H),
                ):
                    descs[(d, s, r)].wait_recv()
                    loc = x_ref[rows, pl.ds(c * NC + off, H)]
                    if s < N_DEV - 2:
                        ref[s + 1, rows, :] = ref[s + 1, rows, :] + loc
                        descs[(d, s + 1, r)].start()
                    else:
                        out_ref[rows, pl.ds(off, H)] = ref[s + 1, rows, :] + loc

        for d in descs.values():
            d.wait_send()

    return pl.pallas_call(
        body,
        out_shape=jax.ShapeDtypeStruct((M, NC), jnp.float32),
        in_specs=[pl.BlockSpec(memory_space=pltpu.VMEM)],
        out_specs=pl.BlockSpec(memory_space=pltpu.VMEM),
        scratch_shapes=[
            pltpu.VMEM((N_DEV, M, H), jnp.float32),
            pltpu.VMEM((N_DEV, M, H), jnp.float32),
            pltpu.SemaphoreType.DMA((N_DEV - 1, K)),
            pltpu.SemaphoreType.DMA((N_DEV - 1, K)),
            pltpu.SemaphoreType.DMA((N_DEV - 1, K)),
            pltpu.SemaphoreType.DMA((N_DEV - 1, K)),
        ],
        compiler_params=pltpu.CompilerParams(collective_id=0),
    )(x2)


# device time: 96963 ns/iter; 1.0124x vs baseline; 1.0124x over previous
import jax
import jax.numpy as jnp
from jax import lax
from jax.experimental import pallas as pl
from jax.experimental.pallas import tpu as pltpu

N_DEV = 16
M = 512
NC = 512
H = NC // 2
K = 2
RS = M // K


def kernel(x):
    x2 = x.reshape(M, N_DEV * NC)

    def body(x_ref, out_ref, cw_ref, ccw_ref,
             cw_send, cw_recv, ccw_send, ccw_recv):
        my = lax.axis_index("i")

        def pos_of(l):
            q = l % 4
            z = l // 4
            return q * 4 + jnp.where(q % 2 == 0, z, 3 - z)

        def id_of(t):
            t = t % N_DEV
            q = t // 4
            zp = t % 4
            z = jnp.where(q % 2 == 0, zp, 3 - zp)
            return z * 4 + q

        p = pos_of(my)
        left = id_of(p - 1)
        right = id_of(p + 1)

        barrier_sem = pltpu.get_barrier_semaphore()
        for nbr in (left, right):
            pl.semaphore_signal(
                barrier_sem, inc=1,
                device_id=(nbr,), device_id_type=pl.DeviceIdType.MESH,
            )
        pl.semaphore_wait(barrier_sem, 2)

        def make(d, s, r):
            ref, ssem, rsem, tgt = (
                (cw_ref, cw_send, cw_recv, right) if d == 0
                else (ccw_ref, ccw_send, ccw_recv, left)
            )
            rows = pl.ds(r * RS, RS)
            if s == 0:
                col = id_of(p - 1) * NC if d == 0 else id_of(p + 1) * NC + H
                src = x_ref.at[rows, pl.ds(col, H)]
            else:
                src = ref.at[s, rows, :]
            return pltpu.make_async_remote_copy(
                src_ref=src,
                dst_ref=ref.at[s + 1, rows, :],
                send_sem=ssem.at[s, r],
                recv_sem=rsem.at[s, r],
                device_id=(tgt,),
                device_id_type=pl.DeviceIdType.MESH,
            )

        descs = {
            (d, s, r): make(d, s, r)
            for d in (0, 1) for s in range(N_DEV - 1) for r in range(K)
        }

        for r in range(K):
            descs[(0, 0, r)].start()
            descs[(1, 0, r)].start()

        for s in range(N_DEV - 1):
            c_cw = id_of(p - 2 - s)
            c_ccw = id_of(p + 2 + s)
            for r in range(K):
                rows = pl.ds(r * RS, RS)
                for d, ref, c, off in (
                    (0, cw_ref, c_cw, 0),
                    (1, ccw_ref, c_ccw, H),
                ):
                    descs[(d, s, r)].wait_recv()
                    loc = x_ref[rows, pl.ds(c * NC + off, H)]
                    if s < N_DEV - 2:
                        ref[s + 1, rows, :] = ref[s + 1, rows, :] + loc
                        descs[(d, s + 1, r)].start()
                    else:
                        out_ref[rows, pl.ds(off, H)] = ref[s + 1, rows, :] + loc

        for d in descs.values():
            d.wait_send()

    return pl.pallas_call(
        body,
        out_shape=jax.ShapeDtypeStruct((M, NC), jnp.float32),
        in_specs=[pl.BlockSpec(memory_space=pltpu.VMEM)],
        out_specs=pl.BlockSpec(memory_space=pltpu.VMEM),
        scratch_shapes=[
            pltpu.VMEM((N_DEV, M, H), jnp.float32),
            pltpu.VMEM((N_DEV, M, H), jnp.float32),
            pltpu.SemaphoreType.DMA((N_DEV - 1, K)),
            pltpu.SemaphoreType.DMA((N_DEV - 1, K)),
            pltpu.SemaphoreType.DMA((N_DEV - 1, K)),
            pltpu.SemaphoreType.DMA((N_DEV - 1, K)),
        ],
        compiler_params=pltpu.CompilerParams(collective_id=0),
    )(x2)
